# baseline (speedup 1.0000x reference)
import functools

import jax
import jax.numpy as jnp
from jax import lax
from jax.experimental import pallas as pl
from jax.experimental.pallas import tpu as pltpu
from jax.experimental.pallas import tpu_sc as plsc

B, H, W = 8, 512, 512
NW = 32
ROWS_PER_W = (B * H) // NW
CHUNK_ROWS = 8
CHUNK_PX = CHUNK_ROWS * W
NCHUNK = ROWS_PER_W // CHUNK_ROWS
GRP = CHUNK_PX // 16
NSTREAM = CHUNK_PX // 128


def _body(x1, gxh, gyh, out, stage, gxb, gyb, idx12, idx22, g12, g22,
          wib, wjb, d1b, ob, dsem):
    wid = lax.axis_index("s") * 2 + lax.axis_index("c")
    pltpu.sync_copy(gxh, gxb)
    pltpu.sync_copy(gyh, gyb)
    wpx0 = wid * (ROWS_PER_W * W)
    b_off = (wid // 4) * (H * W * 4)
    h0w = (wid % 4) * ROWS_PER_W
    lane = lax.iota(jnp.int32, 16)

    def chunk_body(c, _):
        px0 = wpx0 + c * CHUNK_PX
        pltpu.sync_copy(x1.at[pl.ds(px0 * 4, CHUNK_PX * 4)], stage)

        def row_body(r, _):
            h = h0w + c * CHUNK_ROWS + r
            gy_s = plsc.load_gather(gyb, [jnp.full((16,), h, jnp.int32)])

            def grp_body(k, _):
                base = r * W + k * 16
                li0 = base * 4 + lane * 4
                fx = plsc.load_gather(stage, [li0])
                fy = plsc.load_gather(stage, [li0 + 1])
                d1 = plsc.load_gather(stage, [li0 + 2])
                gx_v = gxb[pl.ds(k * 16, 16)]

                i_ = (((gy_s + fy) + 1.0) * 511.0) * 0.5
                j_ = (((gx_v + fx) + 1.0) * 511.0) * 0.5
                icl = jnp.minimum(jnp.maximum(i_, -1.0), 512.0)
                jcl = jnp.minimum(jnp.maximum(j_, -1.0), 512.0)
                ti = icl.astype(jnp.int32)
                tj = jcl.astype(jnp.int32)
                fl_i = ti - jnp.where(ti.astype(jnp.float32) > icl, 1, 0)
                fl_j = tj - jnp.where(tj.astype(jnp.float32) > jcl, 1, 0)
                i_f = jnp.clip(fl_i, 0, H - 1)
                i_c = jnp.clip(fl_i + 1, 0, H - 1)
                j_f = jnp.clip(fl_j, 0, W - 1)
                j_c = jnp.clip(fl_j + 1, 0, W - 1)
                wi = 1.0 - (i_ - i_f.astype(jnp.float32))
                wj = 1.0 - (j_ - j_f.astype(jnp.float32))

                col = j_c * 4 + 3 + b_off
                w12 = i_f * (W * 4) + col
                w22 = i_c * (W * 4) + col

                g = base >> 4
                r2 = g >> 3
                c2 = (g & 7) * 16
                idx12[r2, pl.ds(c2, 16)] = w12
                idx22[r2, pl.ds(c2, 16)] = w22
                wib[pl.ds(base, 16)] = wi
                wjb[pl.ds(base, 16)] = wj
                d1b[pl.ds(base, 16)] = d1
                return 0

            lax.fori_loop(0, W // 16, grp_body, 0)
            return 0

        lax.fori_loop(0, CHUNK_ROWS, row_body, 0)

        def fire(k, _):
            pltpu.make_async_copy(x1.at[idx12.at[k]], g12.at[k], dsem).start()
            pltpu.make_async_copy(x1.at[idx22.at[k]], g22.at[k], dsem).start()
            return 0

        lax.fori_loop(0, NSTREAM, fire, 0)

        def drain(k, _):
            pltpu.make_async_copy(x1.at[idx12.at[k]], g12.at[k], dsem).wait()
            pltpu.make_async_copy(x1.at[idx22.at[k]], g22.at[k], dsem).wait()
            return 0

        lax.fori_loop(0, NSTREAM, drain, 0)

        def comb(g, _):
            q12 = g12[g >> 3, pl.ds((g & 7) * 16, 16)]
            q22 = g22[g >> 3, pl.ds((g & 7) * 16, 16)]
            wi = wib[pl.ds(g * 16, 16)]
            wj = wjb[pl.ds(g * 16, 16)]
            d1 = d1b[pl.ds(g * 16, 16)]
            qi2 = q12 * wi + q22 * (1.0 - wi)
            qij = qi2 * wj + qi2 * (1.0 - wj)
            ob[pl.ds(g * 16, 16)] = qij - d1
            return 0

        lax.fori_loop(0, GRP, comb, 0)
        pltpu.sync_copy(ob, out.at[pl.ds(px0, CHUNK_PX)])
        return 0

    lax.fori_loop(0, NCHUNK, chunk_body, 0)


@jax.jit
def _assoc(x):
    gx = jnp.linspace(-1.0, 1.0, W)
    gy = jnp.linspace(-1.0, 1.0, H)
    x1 = x.reshape(-1)
    run = pl.kernel(
        _body,
        out_type=jax.ShapeDtypeStruct((B * H * W,), jnp.float32),
        mesh=plsc.VectorSubcoreMesh(
            core_axis_name="c", subcore_axis_name="s",
            num_cores=2, num_subcores=16),
        compiler_params=pltpu.CompilerParams(needs_layout_passes=False),
        scratch_types=[
            pltpu.VMEM((CHUNK_PX * 4,), jnp.float32),
            pltpu.VMEM((W,), jnp.float32),
            pltpu.VMEM((H,), jnp.float32),
            pltpu.VMEM((NSTREAM, 128), jnp.int32),
            pltpu.VMEM((NSTREAM, 128), jnp.int32),
            pltpu.VMEM((NSTREAM, 128), jnp.float32),
            pltpu.VMEM((NSTREAM, 128), jnp.float32),
            pltpu.VMEM((CHUNK_PX,), jnp.float32),
            pltpu.VMEM((CHUNK_PX,), jnp.float32),
            pltpu.VMEM((CHUNK_PX,), jnp.float32),
            pltpu.VMEM((CHUNK_PX,), jnp.float32),
            pltpu.SemaphoreType.DMA,
        ],
    )
    return run(x1, gx, gy).reshape(B, H, W, 1)


def kernel(x):
    return _assoc(x)


# speedup vs baseline: 8.2255x; 8.2255x over previous
import functools

import jax
import jax.numpy as jnp
from jax import lax
from jax.experimental import pallas as pl
from jax.experimental.pallas import tpu as pltpu
from jax.experimental.pallas import tpu_sc as plsc

B, H, W = 8, 512, 512
NW = 32
ROWS_PER_W = (B * H) // NW
CHUNK_ROWS = 8
CHUNK_PX = CHUNK_ROWS * W
NCHUNK = ROWS_PER_W // CHUNK_ROWS
GRP = CHUNK_PX // 16
NSTREAM = CHUNK_PX // 128


def _body(xt, d2l, gxh, gyh, out, stage, gxb, gyb, idx12, idx22, g12, g22,
          wib, wjb, d1b, ob, dsem):
    wid = lax.axis_index("s") * 2 + lax.axis_index("c")
    pltpu.sync_copy(gxh, gxb)
    pltpu.sync_copy(gyh, gyb)
    wpx0 = wid * (ROWS_PER_W * W)
    b = wid // 4
    pix_off = b * (H * W)
    h0w = (wid % 4) * ROWS_PER_W
    lane = lax.iota(jnp.int32, 16)

    def chunk_body(c, _):
        px0 = wpx0 + c * CHUNK_PX
        hrow0 = h0w + c * CHUNK_ROWS
        pltpu.sync_copy(xt.at[b, pl.ds(hrow0, CHUNK_ROWS)], stage)

        def row_body(r, _):
            h = hrow0 + r
            gy_s = plsc.load_gather(gyb, [jnp.full((16,), h, jnp.int32)])

            def grp_body(k, _):
                base = r * W + k * 16
                fx = stage[r, 0, pl.ds(k * 16, 16)]
                fy = stage[r, 1, pl.ds(k * 16, 16)]
                d1 = stage[r, 2, pl.ds(k * 16, 16)]
                gx_v = gxb[pl.ds(k * 16, 16)]

                i_ = (((gy_s + fy) + 1.0) * 511.0) * 0.5
                j_ = (((gx_v + fx) + 1.0) * 511.0) * 0.5
                icl = jnp.minimum(jnp.maximum(i_, -1.0), 512.0)
                jcl = jnp.minimum(jnp.maximum(j_, -1.0), 512.0)
                ti = icl.astype(jnp.int32)
                tj = jcl.astype(jnp.int32)
                fl_i = ti - jnp.where(ti.astype(jnp.float32) > icl, 1, 0)
                fl_j = tj - jnp.where(tj.astype(jnp.float32) > jcl, 1, 0)
                i_f = jnp.clip(fl_i, 0, H - 1)
                i_c = jnp.clip(fl_i + 1, 0, H - 1)
                j_f = jnp.clip(fl_j, 0, W - 1)
                j_c = jnp.clip(fl_j + 1, 0, W - 1)
                wi = 1.0 - (i_ - i_f.astype(jnp.float32))
                wj = 1.0 - (j_ - j_f.astype(jnp.float32))

                col = j_c + pix_off
                w12 = i_f * W + col
                w22 = i_c * W + col

                g = base >> 4
                r2 = g >> 3
                c2 = (g & 7) * 16
                idx12[r2, pl.ds(c2, 16)] = w12
                idx22[r2, pl.ds(c2, 16)] = w22
                wib[pl.ds(base, 16)] = wi
                wjb[pl.ds(base, 16)] = wj
                d1b[pl.ds(base, 16)] = d1
                return 0

            lax.fori_loop(0, W // 16, grp_body, 0)
            return 0

        lax.fori_loop(0, CHUNK_ROWS, row_body, 0)

        def fire(k, _):
            pltpu.make_async_copy(d2l.at[idx12.at[k]], g12.at[k], dsem).start()
            pltpu.make_async_copy(d2l.at[idx22.at[k]], g22.at[k], dsem).start()
            return 0

        lax.fori_loop(0, NSTREAM, fire, 0)

        def drain(k, _):
            pltpu.make_async_copy(d2l.at[idx12.at[k]], g12.at[k], dsem).wait()
            pltpu.make_async_copy(d2l.at[idx22.at[k]], g22.at[k], dsem).wait()
            return 0

        lax.fori_loop(0, NSTREAM, drain, 0)

        def comb(g, _):
            q12 = g12[g >> 3, pl.ds((g & 7) * 16, 16)]
            q22 = g22[g >> 3, pl.ds((g & 7) * 16, 16)]
            wi = wib[pl.ds(g * 16, 16)]
            wj = wjb[pl.ds(g * 16, 16)]
            d1 = d1b[pl.ds(g * 16, 16)]
            qi2 = q12 * wi + q22 * (1.0 - wi)
            qij = qi2 * wj + qi2 * (1.0 - wj)
            ob[pl.ds(g * 16, 16)] = qij - d1
            return 0

        lax.fori_loop(0, GRP, comb, 0)
        pltpu.sync_copy(ob, out.at[pl.ds(px0, CHUNK_PX)])
        return 0

    lax.fori_loop(0, NCHUNK, chunk_body, 0)


@jax.jit
def _assoc(x):
    gx = jnp.linspace(-1.0, 1.0, W)
    gy = jnp.linspace(-1.0, 1.0, H)
    xt = jnp.transpose(x, (0, 1, 3, 2))
    d2l = x[..., 3].reshape(-1)
    run = pl.kernel(
        _body,
        out_type=jax.ShapeDtypeStruct((B * H * W,), jnp.float32),
        mesh=plsc.VectorSubcoreMesh(
            core_axis_name="c", subcore_axis_name="s",
            num_cores=2, num_subcores=16),
        compiler_params=pltpu.CompilerParams(needs_layout_passes=False),
        scratch_types=[
            pltpu.VMEM((CHUNK_ROWS, 4, W), jnp.float32),
            pltpu.VMEM((W,), jnp.float32),
            pltpu.VMEM((H,), jnp.float32),
            pltpu.VMEM((NSTREAM, 128), jnp.int32),
            pltpu.VMEM((NSTREAM, 128), jnp.int32),
            pltpu.VMEM((NSTREAM, 128), jnp.float32),
            pltpu.VMEM((NSTREAM, 128), jnp.float32),
            pltpu.VMEM((CHUNK_PX,), jnp.float32),
            pltpu.VMEM((CHUNK_PX,), jnp.float32),
            pltpu.VMEM((CHUNK_PX,), jnp.float32),
            pltpu.VMEM((CHUNK_PX,), jnp.float32),
            pltpu.SemaphoreType.DMA,
        ],
    )
    return run(xt, d2l, gx, gy).reshape(B, H, W, 1)


def kernel(x):
    return _assoc(x)


# speedup vs baseline: 8.2874x; 1.0075x over previous
import functools

import jax
import jax.numpy as jnp
from jax import lax
from jax.experimental import pallas as pl
from jax.experimental.pallas import tpu as pltpu
from jax.experimental.pallas import tpu_sc as plsc

B, H, W = 8, 512, 512
NW = 32
ROWS_PER_W = (B * H) // NW
CHUNK_ROWS = 8
CHUNK_PX = CHUNK_ROWS * W
NCHUNK = ROWS_PER_W // CHUNK_ROWS
GRP = CHUNK_PX // 16
NSTREAM = CHUNK_PX // 128


def _body(xt, d2l, gxh, gyh, out,
          stage0, idx012, idx022, g012, g022, wib0, wjb0, d1b0, ob0,
          stage1, idx112, idx122, g112, g122, wib1, wjb1, d1b1, ob1,
          gxb, gyb, ssem0, gsem0, osem0, ssem1, gsem1, osem1):
    wid = lax.axis_index("s") * 2 + lax.axis_index("c")
    pltpu.sync_copy(gxh, gxb)
    pltpu.sync_copy(gyh, gyb)
    wpx0 = wid * (ROWS_PER_W * W)
    b = wid // 4
    pix_off = b * (H * W)
    h0w = (wid % 4) * ROWS_PER_W
    lane = lax.iota(jnp.int32, 16)

    sets = [
        dict(stage=stage0, i12=idx012, i22=idx022, g12=g012, g22=g022,
             wib=wib0, wjb=wjb0, d1b=d1b0, ob=ob0,
             ssem=ssem0, gsem=gsem0, osem=osem0),
        dict(stage=stage1, i12=idx112, i22=idx122, g12=g112, g22=g122,
             wib=wib1, wjb=wjb1, d1b=d1b1, ob=ob1,
             ssem=ssem1, gsem=gsem1, osem=osem1),
    ]

    def s_copy(c, s):
        return pltpu.make_async_copy(
            xt.at[b, pl.ds(h0w + c * CHUNK_ROWS, CHUNK_ROWS)],
            s["stage"], s["ssem"])

    def o_copy(c, s):
        return pltpu.make_async_copy(
            s["ob"], out.at[pl.ds(wpx0 + c * CHUNK_PX, CHUNK_PX)], s["osem"])

    def p1(c, s):
        stage, i12r, i22r = s["stage"], s["i12"], s["i22"]
        wibr, wjbr, d1br = s["wib"], s["wjb"], s["d1b"]
        hrow0 = h0w + c * CHUNK_ROWS

        def row_body(r, _):
            h = hrow0 + r
            gy_s = plsc.load_gather(gyb, [jnp.full((16,), h, jnp.int32)])

            def grp_body(k, _):
                base = r * W + k * 16
                fx = stage[r, 0, pl.ds(k * 16, 16)]
                fy = stage[r, 1, pl.ds(k * 16, 16)]
                d1 = stage[r, 2, pl.ds(k * 16, 16)]
                gx_v = gxb[pl.ds(k * 16, 16)]

                i_ = (((gy_s + fy) + 1.0) * 511.0) * 0.5
                j_ = (((gx_v + fx) + 1.0) * 511.0) * 0.5
                icl = jnp.minimum(jnp.maximum(i_, -1.0), 512.0)
                jcl = jnp.minimum(jnp.maximum(j_, -1.0), 512.0)
                ti = icl.astype(jnp.int32)
                tj = jcl.astype(jnp.int32)
                fl_i = ti - jnp.where(ti.astype(jnp.float32) > icl, 1, 0)
                fl_j = tj - jnp.where(tj.astype(jnp.float32) > jcl, 1, 0)
                i_f = jnp.clip(fl_i, 0, H - 1)
                i_c = jnp.clip(fl_i + 1, 0, H - 1)
                j_f = jnp.clip(fl_j, 0, W - 1)
                j_c = jnp.clip(fl_j + 1, 0, W - 1)
                wi = 1.0 - (i_ - i_f.astype(jnp.float32))
                wj = 1.0 - (j_ - j_f.astype(jnp.float32))

                col = j_c + pix_off
                g = base >> 4
                r2 = g >> 3
                c2 = (g & 7) * 16
                i12r[r2, pl.ds(c2, 16)] = i_f * W + col
                i22r[r2, pl.ds(c2, 16)] = i_c * W + col
                wibr[pl.ds(base, 16)] = wi
                wjbr[pl.ds(base, 16)] = wj
                d1br[pl.ds(base, 16)] = d1
                return 0

            lax.fori_loop(0, W // 16, grp_body, 0, unroll=2)
            return 0

        lax.fori_loop(0, CHUNK_ROWS, row_body, 0)

    def fire_g(s):
        def fire(k, _):
            pltpu.make_async_copy(
                d2l.at[s["i12"].at[k]], s["g12"].at[k], s["gsem"]).start()
            pltpu.make_async_copy(
                d2l.at[s["i22"].at[k]], s["g22"].at[k], s["gsem"]).start()
            return 0
        lax.fori_loop(0, NSTREAM, fire, 0)

    def wait_g(s):
        def drain(k, _):
            pltpu.make_async_copy(
                d2l.at[s["i12"].at[k]], s["g12"].at[k], s["gsem"]).wait()
            pltpu.make_async_copy(
                d2l.at[s["i22"].at[k]], s["g22"].at[k], s["gsem"]).wait()
            return 0
        lax.fori_loop(0, NSTREAM, drain, 0)

    def p2(s):
        g12r, g22r = s["g12"], s["g22"]
        wibr, wjbr, d1br, obr = s["wib"], s["wjb"], s["d1b"], s["ob"]

        def comb(g, _):
            q12 = g12r[g >> 3, pl.ds((g & 7) * 16, 16)]
            q22 = g22r[g >> 3, pl.ds((g & 7) * 16, 16)]
            wi = wibr[pl.ds(g * 16, 16)]
            wj = wjbr[pl.ds(g * 16, 16)]
            d1 = d1br[pl.ds(g * 16, 16)]
            qi2 = q12 * wi + q22 * (1.0 - wi)
            qij = qi2 * wj + qi2 * (1.0 - wj)
            obr[pl.ds(g * 16, 16)] = qij - d1
            return 0

        lax.fori_loop(0, GRP, comb, 0, unroll=4)

    s_copy(0, sets[0]).start()

    s_copy(0, sets[0]).wait()
    p1(0, sets[0])
    fire_g(sets[0])
    s_copy(1, sets[1]).start()

    def step(c, cur, prv):
        s_copy(c, cur).wait()
        p1(c, cur)
        fire_g(cur)
        s_copy(c + 1, prv).start()
        wait_g(prv)

        @pl.when(c >= 3)
        def _():
            o_copy(c - 3, prv).wait()
        p2(prv)
        o_copy(c - 1, prv).start()

    def loop_body(c2, _):
        step(2 * c2 + 1, sets[1], sets[0])
        step(2 * c2 + 2, sets[0], sets[1])
        return 0

    lax.fori_loop(0, (NCHUNK - 2) // 2, loop_body, 0)

    cL = NCHUNK - 1
    s_copy(cL, sets[1]).wait()
    p1(cL, sets[1])
    fire_g(sets[1])
    wait_g(sets[0])
    o_copy(cL - 3, sets[0]).wait()
    p2(sets[0])
    o_copy(cL - 1, sets[0]).start()
    wait_g(sets[1])
    o_copy(cL - 2, sets[1]).wait()
    p2(sets[1])
    o_copy(cL, sets[1]).start()
    o_copy(cL - 1, sets[0]).wait()
    o_copy(cL, sets[1]).wait()


@jax.jit
def _assoc(x):
    gx = jnp.linspace(-1.0, 1.0, W)
    gy = jnp.linspace(-1.0, 1.0, H)
    xt = jnp.transpose(x, (0, 1, 3, 2))
    d2l = x[..., 3].reshape(-1)

    def bufset():
        return [
            pltpu.VMEM((CHUNK_ROWS, 4, W), jnp.float32),
            pltpu.VMEM((NSTREAM, 128), jnp.int32),
            pltpu.VMEM((NSTREAM, 128), jnp.int32),
            pltpu.VMEM((NSTREAM, 128), jnp.float32),
            pltpu.VMEM((NSTREAM, 128), jnp.float32),
            pltpu.VMEM((CHUNK_PX,), jnp.float32),
            pltpu.VMEM((CHUNK_PX,), jnp.float32),
            pltpu.VMEM((CHUNK_PX,), jnp.float32),
            pltpu.VMEM((CHUNK_PX,), jnp.float32),
        ]

    run = pl.kernel(
        _body,
        out_type=jax.ShapeDtypeStruct((B * H * W,), jnp.float32),
        mesh=plsc.VectorSubcoreMesh(
            core_axis_name="c", subcore_axis_name="s",
            num_cores=2, num_subcores=16),
        compiler_params=pltpu.CompilerParams(needs_layout_passes=False),
        scratch_types=(
            bufset() + bufset() + [
                pltpu.VMEM((W,), jnp.float32),
                pltpu.VMEM((H,), jnp.float32),
            ] + [pltpu.SemaphoreType.DMA] * 6
        ),
    )
    return run(xt, d2l, gx, gy).reshape(B, H, W, 1)


def kernel(x):
    return _assoc(x)


# speedup vs baseline: 8.9007x; 1.0740x over previous
import functools

import jax
import jax.numpy as jnp
from jax import lax
from jax.experimental import pallas as pl
from jax.experimental.pallas import tpu as pltpu
from jax.experimental.pallas import tpu_sc as plsc

B, H, W = 8, 512, 512
NW = 32
ROWS_PER_W = (B * H) // NW
CHUNK_ROWS = 8
CHUNK_PX = CHUNK_ROWS * W
NCHUNK = ROWS_PER_W // CHUNK_ROWS
GRP = CHUNK_PX // 16
NSTREAM = CHUNK_PX // 128


def _body(xt, d2l, gxh, gyh, out,
          stage0, idx012, idx022, g012, g022, wib0, d1b0, ob0,
          stage1, idx112, idx122, g112, g122, wib1, d1b1, ob1,
          gxb, gyb, ssem0, gsem0, osem0, ssem1, gsem1, osem1):
    wid = lax.axis_index("s") * 2 + lax.axis_index("c")
    pltpu.sync_copy(gxh, gxb)
    pltpu.sync_copy(gyh, gyb)
    wpx0 = wid * (ROWS_PER_W * W)
    b = wid // 4
    pix_off = b * (H * W)
    h0w = (wid % 4) * ROWS_PER_W
    lane = lax.iota(jnp.int32, 16)

    sets = [
        dict(stage=stage0, i12=idx012, i22=idx022, g12=g012, g22=g022,
             wib=wib0, d1b=d1b0, ob=ob0,
             ssem=ssem0, gsem=gsem0, osem=osem0),
        dict(stage=stage1, i12=idx112, i22=idx122, g12=g112, g22=g122,
             wib=wib1, d1b=d1b1, ob=ob1,
             ssem=ssem1, gsem=gsem1, osem=osem1),
    ]

    def s_copy(c, s):
        return pltpu.make_async_copy(
            xt.at[b, pl.ds(h0w + c * CHUNK_ROWS, CHUNK_ROWS)],
            s["stage"], s["ssem"])

    def o_copy(c, s):
        return pltpu.make_async_copy(
            s["ob"], out.at[pl.ds(wpx0 + c * CHUNK_PX, CHUNK_PX)], s["osem"])

    def p1(c, s):
        stage, i12r, i22r = s["stage"], s["i12"], s["i22"]
        wibr, d1br = s["wib"], s["d1b"]
        hrow0 = h0w + c * CHUNK_ROWS

        def row_body(r, _):
            h = hrow0 + r
            gy_s = plsc.load_gather(gyb, [jnp.full((16,), h, jnp.int32)])

            def grp_body(k, _):
                base = r * W + k * 16
                fx = stage[r, 0, pl.ds(k * 16, 16)]
                fy = stage[r, 1, pl.ds(k * 16, 16)]
                d1 = stage[r, 2, pl.ds(k * 16, 16)]
                gx_v = gxb[pl.ds(k * 16, 16)]

                i_ = (((gy_s + fy) + 1.0) * 511.0) * 0.5
                j_ = (((gx_v + fx) + 1.0) * 511.0) * 0.5
                icl = jnp.minimum(jnp.maximum(i_, -1.0), 512.0)
                jcl = jnp.minimum(jnp.maximum(j_, -1.0), 512.0)
                ti = icl.astype(jnp.int32)
                tj = jcl.astype(jnp.int32)
                fl_i = ti - jnp.where(ti.astype(jnp.float32) > icl, 1, 0)
                fl_j = tj - jnp.where(tj.astype(jnp.float32) > jcl, 1, 0)
                i_f = jnp.clip(fl_i, 0, H - 1)
                i_c = jnp.clip(fl_i + 1, 0, H - 1)
                j_c = jnp.clip(fl_j + 1, 0, W - 1)
                wi = 1.0 - (i_ - i_f.astype(jnp.float32))

                col = j_c + pix_off
                i12r[pl.ds(base, 16)] = i_f * W + col
                i22r[pl.ds(base, 16)] = i_c * W + col
                wibr[pl.ds(base, 16)] = wi
                d1br[pl.ds(base, 16)] = d1
                return 0

            lax.fori_loop(0, W // 16, grp_body, 0, unroll=4)
            return 0

        lax.fori_loop(0, CHUNK_ROWS, row_body, 0)

    def fire_g(s):
        pltpu.make_async_copy(d2l.at[s["i12"]], s["g12"], s["gsem"]).start()
        pltpu.make_async_copy(d2l.at[s["i22"]], s["g22"], s["gsem"]).start()

    def wait_g(s):
        pltpu.make_async_copy(d2l.at[s["i12"]], s["g12"], s["gsem"]).wait()
        pltpu.make_async_copy(d2l.at[s["i22"]], s["g22"], s["gsem"]).wait()

    def p2(s):
        g12r, g22r = s["g12"], s["g22"]
        wibr, d1br, obr = s["wib"], s["d1b"], s["ob"]

        def comb(g, _):
            q12 = g12r[pl.ds(g * 16, 16)]
            q22 = g22r[pl.ds(g * 16, 16)]
            wi = wibr[pl.ds(g * 16, 16)]
            d1 = d1br[pl.ds(g * 16, 16)]
            qi2 = q12 * wi + q22 * (1.0 - wi)
            obr[pl.ds(g * 16, 16)] = qi2 - d1
            return 0

        lax.fori_loop(0, GRP, comb, 0, unroll=8)

    s_copy(0, sets[0]).start()

    s_copy(0, sets[0]).wait()
    p1(0, sets[0])
    fire_g(sets[0])
    s_copy(1, sets[1]).start()

    def step(c, cur, prv):
        s_copy(c, cur).wait()
        p1(c, cur)
        fire_g(cur)
        s_copy(c + 1, prv).start()
        wait_g(prv)

        @pl.when(c >= 3)
        def _():
            o_copy(c - 3, prv).wait()
        p2(prv)
        o_copy(c - 1, prv).start()

    def loop_body(c2, _):
        step(2 * c2 + 1, sets[1], sets[0])
        step(2 * c2 + 2, sets[0], sets[1])
        return 0

    lax.fori_loop(0, (NCHUNK - 2) // 2, loop_body, 0)

    cL = NCHUNK - 1
    s_copy(cL, sets[1]).wait()
    p1(cL, sets[1])
    fire_g(sets[1])
    wait_g(sets[0])
    o_copy(cL - 3, sets[0]).wait()
    p2(sets[0])
    o_copy(cL - 1, sets[0]).start()
    wait_g(sets[1])
    o_copy(cL - 2, sets[1]).wait()
    p2(sets[1])
    o_copy(cL, sets[1]).start()
    o_copy(cL - 1, sets[0]).wait()
    o_copy(cL, sets[1]).wait()


@jax.jit
def _assoc(x):
    gx = jnp.linspace(-1.0, 1.0, W)
    gy = jnp.linspace(-1.0, 1.0, H)
    xt = jnp.transpose(x, (0, 1, 3, 2))
    d2l = x[..., 3].reshape(-1)

    def bufset():
        return [
            pltpu.VMEM((CHUNK_ROWS, 4, W), jnp.float32),
            pltpu.VMEM((CHUNK_PX,), jnp.int32),
            pltpu.VMEM((CHUNK_PX,), jnp.int32),
            pltpu.VMEM((CHUNK_PX,), jnp.float32),
            pltpu.VMEM((CHUNK_PX,), jnp.float32),
            pltpu.VMEM((CHUNK_PX,), jnp.float32),
            pltpu.VMEM((CHUNK_PX,), jnp.float32),
            pltpu.VMEM((CHUNK_PX,), jnp.float32),
        ]

    run = pl.kernel(
        _body,
        out_type=jax.ShapeDtypeStruct((B * H * W,), jnp.float32),
        mesh=plsc.VectorSubcoreMesh(
            core_axis_name="c", subcore_axis_name="s",
            num_cores=2, num_subcores=16),
        compiler_params=pltpu.CompilerParams(needs_layout_passes=False),
        scratch_types=(
            bufset() + bufset() + [
                pltpu.VMEM((W,), jnp.float32),
                pltpu.VMEM((H,), jnp.float32),
            ] + [pltpu.SemaphoreType.DMA] * 6
        ),
    )
    return run(xt, d2l, gx, gy).reshape(B, H, W, 1)


def kernel(x):
    return _assoc(x)


# speedup vs baseline: 11.1868x; 1.2568x over previous
import functools

import jax
import jax.numpy as jnp
from jax import lax
from jax.experimental import pallas as pl
from jax.experimental.pallas import tpu as pltpu
from jax.experimental.pallas import tpu_sc as plsc

B, H, W = 8, 512, 512
NW = 32
ROWS_PER_W = (B * H) // NW
CHUNK_ROWS = 4
CHUNK_PX = CHUNK_ROWS * W
NCHUNK = ROWS_PER_W // CHUNK_ROWS
GRP = CHUNK_PX // 16
NSTREAM = CHUNK_PX // 128


def _body(xt, d2l, gxh, gyh, out,
          stage0, idx012, idx022, g012, g022, wib0, d1b0, ob0,
          stage1, idx112, idx122, g112, g122, wib1, d1b1, ob1,
          gxb, gyb, tbl, ssem0, gsem0, osem0, ssem1, gsem1, osem1):
    cax = lax.axis_index("c")
    sax = lax.axis_index("s")
    wid = cax * 16 + sax
    pltpu.sync_copy(gxh, gxb)
    pltpu.sync_copy(gyh, gyb)
    tslice = 4 * H * W // 16
    pltpu.sync_copy(d2l.at[pl.ds(cax * (4 * H * W) + sax * tslice, tslice)],
                    tbl.at[pl.ds(sax * tslice, tslice)])
    plsc.subcore_barrier()
    wpx0 = wid * (ROWS_PER_W * W)
    b = wid // 4
    pix_off = (sax // 4) * (H * W)
    h0w = (wid % 4) * ROWS_PER_W
    lane = lax.iota(jnp.int32, 16)

    sets = [
        dict(stage=stage0, i12=idx012, i22=idx022, g12=g012, g22=g022,
             wib=wib0, d1b=d1b0, ob=ob0,
             ssem=ssem0, gsem=gsem0, osem=osem0),
        dict(stage=stage1, i12=idx112, i22=idx122, g12=g112, g22=g122,
             wib=wib1, d1b=d1b1, ob=ob1,
             ssem=ssem1, gsem=gsem1, osem=osem1),
    ]

    def s_copy(c, s):
        return pltpu.make_async_copy(
            xt.at[b, pl.ds(h0w + c * CHUNK_ROWS, CHUNK_ROWS)],
            s["stage"], s["ssem"])

    def o_copy(c, s):
        return pltpu.make_async_copy(
            s["ob"], out.at[pl.ds(wpx0 + c * CHUNK_PX, CHUNK_PX)], s["osem"])

    def p1(c, s):
        stage, i12r, i22r = s["stage"], s["i12"], s["i22"]
        wibr, d1br = s["wib"], s["d1b"]
        hrow0 = h0w + c * CHUNK_ROWS

        def row_body(r, _):
            h = hrow0 + r
            gy_s = plsc.load_gather(gyb, [jnp.full((16,), h, jnp.int32)])

            def grp_body(k, _):
                base = r * W + k * 16
                fx = stage[r, 0, pl.ds(k * 16, 16)]
                fy = stage[r, 1, pl.ds(k * 16, 16)]
                d1 = stage[r, 2, pl.ds(k * 16, 16)]
                gx_v = gxb[pl.ds(k * 16, 16)]

                i_ = (((gy_s + fy) + 1.0) * 511.0) * 0.5
                j_ = (((gx_v + fx) + 1.0) * 511.0) * 0.5
                icl = jnp.minimum(jnp.maximum(i_, -1.0), 512.0)
                jcl = jnp.minimum(jnp.maximum(j_, -1.0), 512.0)
                ti = icl.astype(jnp.int32)
                tj = jcl.astype(jnp.int32)
                fl_i = ti - jnp.where(ti.astype(jnp.float32) > icl, 1, 0)
                fl_j = tj - jnp.where(tj.astype(jnp.float32) > jcl, 1, 0)
                i_f = jnp.clip(fl_i, 0, H - 1)
                i_c = jnp.clip(fl_i + 1, 0, H - 1)
                j_c = jnp.clip(fl_j + 1, 0, W - 1)
                wi = 1.0 - (i_ - i_f.astype(jnp.float32))

                col = j_c + pix_off
                i12r[pl.ds(base, 16)] = i_f * W + col
                i22r[pl.ds(base, 16)] = i_c * W + col
                wibr[pl.ds(base, 16)] = wi
                d1br[pl.ds(base, 16)] = d1
                return 0

            lax.fori_loop(0, W // 16, grp_body, 0, unroll=4)
            return 0

        lax.fori_loop(0, CHUNK_ROWS, row_body, 0)

    def fire_g(s):
        pltpu.make_async_copy(tbl.at[s["i12"]], s["g12"], s["gsem"]).start()
        pltpu.make_async_copy(tbl.at[s["i22"]], s["g22"], s["gsem"]).start()

    def wait_g(s):
        pltpu.make_async_copy(tbl.at[s["i12"]], s["g12"], s["gsem"]).wait()
        pltpu.make_async_copy(tbl.at[s["i22"]], s["g22"], s["gsem"]).wait()

    def p2(s):
        g12r, g22r = s["g12"], s["g22"]
        wibr, d1br, obr = s["wib"], s["d1b"], s["ob"]

        def comb(g, _):
            q12 = g12r[pl.ds(g * 16, 16)]
            q22 = g22r[pl.ds(g * 16, 16)]
            wi = wibr[pl.ds(g * 16, 16)]
            d1 = d1br[pl.ds(g * 16, 16)]
            qi2 = q12 * wi + q22 * (1.0 - wi)
            obr[pl.ds(g * 16, 16)] = qi2 - d1
            return 0

        lax.fori_loop(0, GRP, comb, 0, unroll=8)

    s_copy(0, sets[0]).start()

    s_copy(0, sets[0]).wait()
    p1(0, sets[0])
    fire_g(sets[0])
    s_copy(1, sets[1]).start()

    def step(c, cur, prv):
        s_copy(c, cur).wait()
        p1(c, cur)
        fire_g(cur)
        s_copy(c + 1, prv).start()
        wait_g(prv)

        @pl.when(c >= 3)
        def _():
            o_copy(c - 3, prv).wait()
        p2(prv)
        o_copy(c - 1, prv).start()

    def loop_body(c2, _):
        step(2 * c2 + 1, sets[1], sets[0])
        step(2 * c2 + 2, sets[0], sets[1])
        return 0

    lax.fori_loop(0, (NCHUNK - 2) // 2, loop_body, 0)

    cL = NCHUNK - 1
    s_copy(cL, sets[1]).wait()
    p1(cL, sets[1])
    fire_g(sets[1])
    wait_g(sets[0])
    o_copy(cL - 3, sets[0]).wait()
    p2(sets[0])
    o_copy(cL - 1, sets[0]).start()
    wait_g(sets[1])
    o_copy(cL - 2, sets[1]).wait()
    p2(sets[1])
    o_copy(cL, sets[1]).start()
    o_copy(cL - 1, sets[0]).wait()
    o_copy(cL, sets[1]).wait()


@jax.jit
def _assoc(x):
    gx = jnp.linspace(-1.0, 1.0, W)
    gy = jnp.linspace(-1.0, 1.0, H)
    xt = jnp.transpose(x, (0, 1, 3, 2))
    d2l = x[..., 3].reshape(-1)

    def bufset():
        return [
            pltpu.VMEM((CHUNK_ROWS, 4, W), jnp.float32),
            pltpu.VMEM((CHUNK_PX,), jnp.int32),
            pltpu.VMEM((CHUNK_PX,), jnp.int32),
            pltpu.VMEM((CHUNK_PX,), jnp.float32),
            pltpu.VMEM((CHUNK_PX,), jnp.float32),
            pltpu.VMEM((CHUNK_PX,), jnp.float32),
            pltpu.VMEM((CHUNK_PX,), jnp.float32),
            pltpu.VMEM((CHUNK_PX,), jnp.float32),
        ]

    run = pl.kernel(
        _body,
        out_type=jax.ShapeDtypeStruct((B * H * W,), jnp.float32),
        mesh=plsc.VectorSubcoreMesh(
            core_axis_name="c", subcore_axis_name="s",
            num_cores=2, num_subcores=16),
        compiler_params=pltpu.CompilerParams(needs_layout_passes=False),
        scratch_types=(
            bufset() + bufset() + [
                pltpu.VMEM((W,), jnp.float32),
                pltpu.VMEM((H,), jnp.float32),
                pltpu.VMEM_SHARED((4 * H * W,), jnp.float32),
            ] + [pltpu.SemaphoreType.DMA] * 6
        ),
    )
    return run(xt, d2l, gx, gy).reshape(B, H, W, 1)


def kernel(x):
    return _assoc(x)
